# Initial kernel scaffold; baseline (speedup 1.0000x reference)
import jax
import jax.numpy as jnp
from jax.experimental import pallas as pl


def kernel(x, adaptive_params, W1, b1, W2, b2):
    raise NotImplementedError("write your pallas kernel here")


# speedup vs baseline: 3268.9722x; 3268.9722x over previous
import jax
import jax.numpy as jnp
from jax.experimental import pallas as pl


def _gcn_fused_kernel(x_ref, p_ref, w1_ref, b1_ref, w2_ref, b2_ref, out_ref):
    p = p_ref[...]
    n = p.shape[0]
    row_i = jax.lax.broadcasted_iota(jnp.int32, (n, n), 0)
    col_i = jax.lax.broadcasted_iota(jnp.int32, (n, n), 1)
    eye = jnp.where(row_i == col_i, jnp.float32(1.0), jnp.float32(0.0))
    a = jax.nn.sigmoid(eye + 0.5 * (p + p.T))
    dis_c = jax.lax.rsqrt(jnp.sum(a, axis=0, keepdims=True))
    dis_r = jax.lax.rsqrt(jnp.sum(a, axis=1, keepdims=True))
    a_hat = a * dis_c * dis_r

    xw = jnp.dot(x_ref[...], w1_ref[...], preferred_element_type=jnp.float32)
    h = jnp.maximum(
        jnp.dot(a_hat, xw, preferred_element_type=jnp.float32) + b1_ref[...],
        0.0,
    )
    hw = jnp.dot(h, w2_ref[...], preferred_element_type=jnp.float32)
    out_ref[...] = (
        jnp.dot(a_hat, hw, preferred_element_type=jnp.float32) + b2_ref[...]
    )


@jax.jit
def kernel(x, adaptive_params, W1, b1, W2, b2):
    n = x.shape[0]
    return pl.pallas_call(
        _gcn_fused_kernel,
        out_shape=jax.ShapeDtypeStruct((n, W2.shape[1]), x.dtype),
    )(x, adaptive_params, W1, b1.reshape(1, -1), W2, b2.reshape(1, -1))
